# baseline (speedup 1.0000x reference)
import functools

import jax
import jax.numpy as jnp
from jax import lax
from jax.experimental import pallas as pl
from jax.experimental.pallas import tpu as pltpu
from jax.experimental.pallas import tpu_sc as plsc

BATCH = 4096
SEQ_LEN = 200
VOCAB = 1000000
EMBED = 32

_INFO = plsc.get_sparse_core_info()
NC = _INFO.num_cores
NS = _INFO.num_subcores
NW = NC * NS
B = BATCH * SEQ_LEN
B_PER_W = B // NW
IDXW = 128
CHUNK = 1280
NCH = CHUNK // IDXW
N_CHUNKS = B_PER_W // CHUNK
N_OUTER = N_CHUNKS // 2


def _make_kernel():
    mesh = plsc.VectorSubcoreMesh(core_axis_name="c", subcore_axis_name="s")

    @functools.partial(
        pl.kernel,
        mesh=mesh,
        out_type=jax.ShapeDtypeStruct((B, EMBED), jnp.float32),
        compiler_params=pltpu.CompilerParams(use_tc_tiling_on_sc=False),
        scratch_types=[
            pltpu.VMEM((NCH, IDXW), jnp.int32),
            pltpu.VMEM((NCH, IDXW), jnp.int32),
            pltpu.VMEM((CHUNK, EMBED), jnp.float32),
            pltpu.VMEM((CHUNK, EMBED), jnp.float32),
            pltpu.SemaphoreType.DMA,
            pltpu.SemaphoreType.DMA,
            pltpu.SemaphoreType.DMA,
        ],
    )
    def k(idx_hbm, table_hbm, out_hbm, idx0, idx1, rows0, rows1,
          sem_g, sem_o0, sem_o1):
        wid = lax.axis_index("s") * NC + lax.axis_index("c")
        base = wid * B_PER_W
        row_base = wid * (B_PER_W // IDXW)

        bufs = ((idx0, rows0, sem_o0), (idx1, rows1, sem_o1))

        def body(i, carry):
            for h, (idx_v, rows_v, sem_o) in enumerate(bufs):
                c = 2 * i + h

                @pl.when(i > 0)
                def _():
                    pltpu.make_async_copy(
                        rows_v, out_hbm.at[pl.ds(base, CHUNK)], sem_o
                    ).wait()

                pltpu.sync_copy(
                    idx_hbm.at[pl.ds(row_base + c * NCH, NCH)], idx_v
                )
                copies = [
                    pltpu.async_copy(
                        table_hbm.at[idx_v.at[j]],
                        rows_v.at[pl.ds(j * IDXW, IDXW)],
                        sem_g,
                    )
                    for j in range(NCH)
                ]
                for cp in copies:
                    cp.wait()
                pltpu.async_copy(
                    rows_v, out_hbm.at[pl.ds(base + c * CHUNK, CHUNK)], sem_o
                )
            return carry

        lax.fori_loop(0, N_OUTER, body, 0)

        for _, rows_v, sem_o in bufs:
            pltpu.make_async_copy(
                rows_v, out_hbm.at[pl.ds(base, CHUNK)], sem_o
            ).wait()

    return k


_kernel_call = _make_kernel()


def kernel(x, table):
    idx = x.reshape(B // IDXW, IDXW)
    out = _kernel_call(idx, table)
    return out.reshape(BATCH, SEQ_LEN, EMBED)


# speedup vs baseline: 1.0498x; 1.0498x over previous
import functools

import jax
import jax.numpy as jnp
from jax import lax
from jax.experimental import pallas as pl
from jax.experimental.pallas import tpu as pltpu
from jax.experimental.pallas import tpu_sc as plsc

BATCH = 4096
SEQ_LEN = 200
VOCAB = 1000000
EMBED = 32

_INFO = plsc.get_sparse_core_info()
NC = _INFO.num_cores
NS = _INFO.num_subcores
NW = NC * NS
B = BATCH * SEQ_LEN
B_PER_W = B // NW
IDXW = 128
CHUNK = 1280
NCH = CHUNK // IDXW
N_CHUNKS = B_PER_W // CHUNK
N_OUTER = N_CHUNKS // 2


def _make_kernel():
    mesh = plsc.VectorSubcoreMesh(core_axis_name="c", subcore_axis_name="s")

    @functools.partial(
        pl.kernel,
        mesh=mesh,
        out_type=jax.ShapeDtypeStruct((B, EMBED), jnp.float32),
        compiler_params=pltpu.CompilerParams(use_tc_tiling_on_sc=False),
        scratch_types=[
            pltpu.VMEM((NCH, IDXW), jnp.int32),
            pltpu.VMEM((NCH, IDXW), jnp.int32),
            pltpu.VMEM((CHUNK, EMBED), jnp.float32),
            pltpu.VMEM((CHUNK, EMBED), jnp.float32),
            pltpu.SemaphoreType.DMA,
            pltpu.SemaphoreType.DMA,
            pltpu.SemaphoreType.DMA,
        ],
    )
    def k(idx_hbm, table_hbm, out_hbm, idx0, idx1, rows0, rows1,
          sem_g, sem_o0, sem_o1):
        wid = lax.axis_index("s") * NC + lax.axis_index("c")
        base = wid * B_PER_W
        row_base = wid * (B_PER_W // IDXW)

        bufs = ((idx0, rows0, sem_o0), (idx1, rows1, sem_o1))

        def body(i, carry):
            for h, (idx_v, rows_v, sem_o) in enumerate(bufs):
                c = 2 * i + h

                @pl.when(i > 0)
                def _():
                    pltpu.make_async_copy(
                        rows_v, out_hbm.at[pl.ds(base, CHUNK)], sem_o
                    ).wait()

                pltpu.sync_copy(
                    idx_hbm.at[pl.ds(row_base + c * NCH, NCH)], idx_v
                )
                copies = [
                    pltpu.async_copy(
                        table_hbm.at[idx_v.at[j]],
                        rows_v.at[pl.ds(j * IDXW, IDXW)],
                        sem_g,
                    )
                    for j in range(NCH)
                ]
                for cp in copies:
                    cp.wait()
                pltpu.async_copy(
                    rows_v, out_hbm.at[pl.ds(base + c * CHUNK, CHUNK)], sem_o
                )
            return carry

        lax.fori_loop(0, N_OUTER, body, 0)

        for _, rows_v, sem_o in bufs:
            pltpu.make_async_copy(
                rows_v, out_hbm.at[pl.ds(base, CHUNK)], sem_o
            ).wait()

    return k


_kernel_call = _make_kernel()


def kernel(x, table):
    idx = jnp.transpose(jnp.squeeze(x, -1)).reshape(B // IDXW, IDXW)
    out = _kernel_call(idx, table)
    return jnp.transpose(out.reshape(SEQ_LEN, BATCH, EMBED), (1, 0, 2))


# speedup vs baseline: 1.3931x; 1.3270x over previous
import functools

import jax
import jax.numpy as jnp
from jax import lax
from jax.experimental import pallas as pl
from jax.experimental.pallas import tpu as pltpu
from jax.experimental.pallas import tpu_sc as plsc

BATCH = 4096
SEQ_LEN = 200
VOCAB = 1000000
EMBED = 32

_INFO = plsc.get_sparse_core_info()
NC = _INFO.num_cores
NS = _INFO.num_subcores
NW = NC * NS
B = BATCH * SEQ_LEN
B_PER_W = B // NW
IDXW = 128
N_SUPER = B_PER_W // (8 * IDXW)
TPB = BATCH // IDXW


def _make_kernel():
    mesh = plsc.VectorSubcoreMesh(core_axis_name="c", subcore_axis_name="s")

    @functools.partial(
        pl.kernel,
        mesh=mesh,
        out_type=jax.ShapeDtypeStruct(
            (SEQ_LEN, EMBED // 8, TPB, 8, IDXW), jnp.float32
        ),
        compiler_params=pltpu.CompilerParams(
            use_tc_tiling_on_sc=False, needs_layout_passes=False
        ),
        scratch_types=[
            pltpu.VMEM((8, IDXW), jnp.int32),
            pltpu.VMEM((IDXW, EMBED), jnp.float32),
            pltpu.VMEM((IDXW, EMBED), jnp.float32),
            pltpu.VMEM((EMBED, IDXW + 1), jnp.float32),
            pltpu.VMEM((EMBED, IDXW + 1), jnp.float32),
            pltpu.SemaphoreType.DMA,
            pltpu.SemaphoreType.DMA,
            pltpu.SemaphoreType.DMA,
        ],
    )
    def k(idx_hbm, table_hbm, out_hbm, idx_v, rows0, rows1, tt0, tt1,
          sem_g, sem_o0, sem_o1):
        wid = lax.axis_index("s") * NC + lax.axis_index("c")
        base = wid * B_PER_W
        row_base = wid * (B_PER_W // IDXW)

        rows = (rows0, rows1)
        tts = ((tt0, sem_o0), (tt1, sem_o1))
        iota16 = lax.iota(jnp.int32, 16)

        def outer(s8, carry):
            pltpu.sync_copy(
                idx_hbm.at[pl.ds(pl.multiple_of(row_base + s8 * 8, 8), 8)],
                idx_v,
            )
            handles = [
                pltpu.async_copy(
                    table_hbm.at[idx_v.at[0]], rows[0], sem_g
                )
            ]
            for r in range(8):
                handles[r].wait()
                if r < 7:
                    handles.append(
                        pltpu.async_copy(
                            table_hbm.at[idx_v.at[r + 1]],
                            rows[(r + 1) % 2],
                            sem_g,
                        )
                    )
                rows_v = rows[r % 2]
                tt_v, sem_o = tts[r % 2]

                def drain():
                    for _ in range(4):
                        pltpu.make_async_copy(
                            tt_v.at[pl.ds(0, 8), pl.ds(0, IDXW)],
                            out_hbm.at[0, 0, 0],
                            sem_o,
                        ).wait()

                if r < 2:
                    @pl.when(s8 > 0)
                    def _():
                        drain()
                else:
                    drain()

                def tb(b, c):
                    colb = jnp.zeros((16,), jnp.int32) + b
                    v0 = rows_v[b, pl.ds(0, 16)]
                    v1 = rows_v[b, pl.ds(16, 16)]
                    plsc.store_scatter(tt_v, [iota16, colb], v0)
                    plsc.store_scatter(tt_v, [iota16 + 16, colb], v1)
                    return c

                lax.fori_loop(0, IDXW, tb, 0)

                p0 = base + (s8 * 8 + r) * IDXW
                s_pos = p0 // BATCH
                t_pos = (p0 % BATCH) // IDXW
                for g in range(4):
                    pltpu.async_copy(
                        tt_v.at[pl.ds(8 * g, 8), pl.ds(0, IDXW)],
                        out_hbm.at[s_pos, g, t_pos],
                        sem_o,
                    )
            return carry

        lax.fori_loop(0, N_SUPER, outer, 0)

        for tt_v, sem_o in tts:
            for _ in range(4):
                pltpu.make_async_copy(
                    tt_v.at[pl.ds(0, 8), pl.ds(0, IDXW)],
                    out_hbm.at[0, 0, 0],
                    sem_o,
                ).wait()

    return k


_kernel_call = _make_kernel()


def kernel(x, table):
    idx = jnp.transpose(jnp.squeeze(x, -1)).reshape(B // IDXW, IDXW)
    out5 = _kernel_call(idx, table)
    return jnp.transpose(out5, (2, 4, 0, 1, 3)).reshape(BATCH, SEQ_LEN, EMBED)


# speedup vs baseline: 1.4193x; 1.0188x over previous
import functools

import jax
import jax.numpy as jnp
from jax import lax
from jax.experimental import pallas as pl
from jax.experimental.pallas import tpu as pltpu
from jax.experimental.pallas import tpu_sc as plsc

BATCH = 4096
SEQ_LEN = 200
VOCAB = 1000000
EMBED = 32

_INFO = plsc.get_sparse_core_info()
NC = _INFO.num_cores
NS = _INFO.num_subcores
NW = NC * NS
B = BATCH * SEQ_LEN
B_PER_W = B // NW
IDXW = 128
N_SUPER = B_PER_W // (8 * IDXW)
TPB = BATCH // IDXW


def _make_kernel():
    mesh = plsc.VectorSubcoreMesh(core_axis_name="c", subcore_axis_name="s")

    @functools.partial(
        pl.kernel,
        mesh=mesh,
        out_type=jax.ShapeDtypeStruct(
            (SEQ_LEN, EMBED // 8, TPB, 8, IDXW), jnp.float32
        ),
        compiler_params=pltpu.CompilerParams(
            use_tc_tiling_on_sc=False, needs_layout_passes=False
        ),
        scratch_types=[
            pltpu.VMEM((8, IDXW), jnp.int32),
            pltpu.VMEM((8, IDXW), jnp.int32),
            pltpu.VMEM((IDXW, EMBED), jnp.float32),
            pltpu.VMEM((IDXW, EMBED), jnp.float32),
            pltpu.VMEM((EMBED, IDXW + 1), jnp.float32),
            pltpu.VMEM((EMBED, IDXW + 1), jnp.float32),
            pltpu.SemaphoreType.DMA,
            pltpu.SemaphoreType.DMA,
            pltpu.SemaphoreType.DMA,
            pltpu.SemaphoreType.DMA,
        ],
    )
    def k(idx_hbm, table_hbm, out_hbm, idxA, idxB, rows0, rows1, tt0, tt1,
          sem_g, sem_i, sem_o0, sem_o1):
        wid = lax.axis_index("s") * NC + lax.axis_index("c")
        base = wid * B_PER_W
        row_base = wid * (B_PER_W // IDXW)

        rows = (rows0, rows1)
        tts = ((tt0, sem_o0), (tt1, sem_o1))
        iota16 = lax.iota(jnp.int32, 16)

        def idx_src(s8):
            return idx_hbm.at[
                pl.ds(pl.multiple_of(row_base + s8 * 8, 8), 8)
            ]

        def do_super(s8, idx_cur, idx_nxt, first, prefetch):
            if prefetch:
                pltpu.async_copy(idx_src(s8 + 1), idx_nxt, sem_i)
            for r in range(8):
                pltpu.make_async_copy(
                    table_hbm.at[idx_cur.at[r]], rows[r % 2], sem_g
                ).wait()
                if r < 7:
                    pltpu.async_copy(
                        table_hbm.at[idx_cur.at[r + 1]],
                        rows[(r + 1) % 2],
                        sem_g,
                    )
                rows_v = rows[r % 2]
                tt_v, sem_o = tts[r % 2]

                def drain():
                    for _ in range(4):
                        pltpu.make_async_copy(
                            tt_v.at[pl.ds(0, 8), pl.ds(0, IDXW)],
                            out_hbm.at[0, 0, 0],
                            sem_o,
                        ).wait()

                if r < 2:
                    if first is not True:
                        @pl.when(jnp.logical_not(first))
                        def _():
                            drain()
                else:
                    drain()

                def tb(i, c):
                    for db in range(4):
                        b = 4 * i + db
                        colb = jnp.zeros((16,), jnp.int32) + b
                        v0 = rows_v[b, pl.ds(0, 16)]
                        v1 = rows_v[b, pl.ds(16, 16)]
                        plsc.store_scatter(tt_v, [iota16, colb], v0)
                        plsc.store_scatter(tt_v, [iota16 + 16, colb], v1)
                    return c

                lax.fori_loop(0, IDXW // 4, tb, 0)

                p0 = base + (s8 * 8 + r) * IDXW
                s_pos = p0 // BATCH
                t_pos = (p0 % BATCH) // IDXW
                for g in range(4):
                    pltpu.async_copy(
                        tt_v.at[pl.ds(8 * g, 8), pl.ds(0, IDXW)],
                        out_hbm.at[s_pos, g, t_pos],
                        sem_o,
                    )
            if prefetch:
                pltpu.make_async_copy(idx_src(s8 + 1), idx_nxt, sem_i).wait()
                pltpu.async_copy(
                    table_hbm.at[idx_nxt.at[0]], rows[0], sem_g
                )

        pltpu.sync_copy(idx_src(0), idxA)
        pltpu.async_copy(table_hbm.at[idxA.at[0]], rows[0], sem_g)

        def body(i, carry):
            do_super(2 * i, idxA, idxB, i == 0, True)
            do_super(2 * i + 1, idxB, idxA, False, True)
            return carry

        lax.fori_loop(0, (N_SUPER - 1) // 2, body, 0)
        do_super(N_SUPER - 1, idxA, idxB, False, False)

        for tt_v, sem_o in tts:
            for _ in range(4):
                pltpu.make_async_copy(
                    tt_v.at[pl.ds(0, 8), pl.ds(0, IDXW)],
                    out_hbm.at[0, 0, 0],
                    sem_o,
                ).wait()

    return k


_kernel_call = _make_kernel()


def kernel(x, table):
    idx = jnp.transpose(jnp.squeeze(x, -1)).reshape(B // IDXW, IDXW)
    out5 = _kernel_call(idx, table)
    return jnp.transpose(out5, (2, 4, 0, 1, 3)).reshape(BATCH, SEQ_LEN, EMBED)
